# baseline (speedup 1.0000x reference)
import functools

import jax
import jax.numpy as jnp
from jax import lax
from jax.experimental import pallas as pl
from jax.experimental.pallas import tpu as pltpu
from jax.experimental.pallas import tpu_sc as plsc

B = 4096
F = 26
V = 1000000
D = 16
NUM = 13
H1 = 256
H2 = 128
EPS = 1e-5

NC = 2
NS = 16
NW = NC * NS
NP = F * D
PPW = NP // NW
NST = B // 128

_mesh = plsc.VectorSubcoreMesh(core_axis_name="c", subcore_axis_name="s")


@functools.partial(
    pl.kernel,
    mesh=_mesh,
    out_type=jax.ShapeDtypeStruct((NP * B,), jnp.float32),
    scratch_types=[
        pltpu.VMEM((2, NST, 128), jnp.int32),
        pltpu.VMEM((2, B), jnp.float32),
        pltpu.SemaphoreType.DMA,
        pltpu.SemaphoreType.DMA,
    ],
    compiler_params=pltpu.CompilerParams(
        use_tc_tiling_on_sc=False, needs_layout_passes=False),
)
def _sc_gather(table_hbm, xcat_hbm, out_hbm, idx_v, ring_v, sem, osem):
    wid = lax.axis_index("s") * NC + lax.axis_index("c")
    p0 = wid * PPW
    f0 = p0 >> 4
    f1 = jnp.minimum(f0 + 1, F - 1)
    pltpu.sync_copy(xcat_hbm.at[f0], idx_v.at[0])
    pltpu.sync_copy(xcat_hbm.at[f1], idx_v.at[1])

    def fire(k):
        p = p0 + k
        f = p >> 4
        d = p & 15
        fsel = f - f0
        slot = k % 2
        for c in range(NST):
            pltpu.make_async_copy(
                table_hbm.at[f, d].at[idx_v.at[fsel, c]],
                ring_v.at[slot, pl.ds(c * 128, 128)],
                sem,
            ).start()

    def drain(k):
        slot = k % 2
        for c in range(NST):
            pltpu.make_async_copy(
                table_hbm.at[0, 0].at[idx_v.at[0, c]],
                ring_v.at[slot, pl.ds(c * 128, 128)],
                sem,
            ).wait()
        pltpu.make_async_copy(
            ring_v.at[slot], out_hbm.at[pl.ds((p0 + k) * B, B)], osem,
        ).start()

    fire(0)
    for k in range(1, PPW):
        if k >= 2:
            pltpu.make_async_copy(
                ring_v.at[0], out_hbm.at[pl.ds(0, B)], osem).wait()
        fire(k)
        drain(k - 1)
    drain(PPW - 1)
    pltpu.make_async_copy(
        ring_v.at[0], out_hbm.at[pl.ds(0, B)], osem).wait()
    pltpu.make_async_copy(
        ring_v.at[0], out_hbm.at[pl.ds(0, B)], osem).wait()


def _mlp_body(xg_ref, xn_ref,
              bg_ref, bb_ref, bm_ref, bv_ref,
              w0c_ref, w0n_ref, b0_ref, g0_ref, be0_ref, m0_ref, v0_ref,
              w1_ref, b1_ref, g1_ref, be1_ref, m1_ref, v1_ref,
              w2_ref, b2_ref, out_ref):
    xn = xn_ref[...]
    xnb = (xn - bm_ref[...]) * lax.rsqrt(bv_ref[...] + EPS) * bg_ref[...] + bb_ref[...]
    h = lax.dot_general(xg_ref[...], w0c_ref[...], (((0,), (0,)), ((), ())),
                        preferred_element_type=jnp.float32)
    h = h + jnp.dot(xnb, w0n_ref[...], preferred_element_type=jnp.float32)
    h = jnp.maximum(h + b0_ref[...], 0.0)
    h = (h - m0_ref[...]) * lax.rsqrt(v0_ref[...] + EPS) * g0_ref[...] + be0_ref[...]
    h = jnp.dot(h, w1_ref[...], preferred_element_type=jnp.float32)
    h = jnp.maximum(h + b1_ref[...], 0.0)
    h = (h - m1_ref[...]) * lax.rsqrt(v1_ref[...] + EPS) * g1_ref[...] + be1_ref[...]
    out_ref[...] = jnp.dot(h, w2_ref[...], preferred_element_type=jnp.float32) + b2_ref[...]


def _tc_mlp(xg, xn, bg, bb, bm, bv, w0c, w0n, b0, g0, be0, m0, v0,
            w1, b1, g1, be1, m1, v1, w2, b2):
    TB = 512
    grid = (B // TB,)
    col = lambda i: (0, i)
    row = lambda i: (i, 0)
    rep = lambda i: (0, 0)
    full = lambda a: pl.BlockSpec(a.shape, rep)
    return pl.pallas_call(
        _mlp_body,
        grid=grid,
        in_specs=[
            pl.BlockSpec((NP, TB), col),
            pl.BlockSpec((TB, NUM), row),
            full(bg), full(bb), full(bm), full(bv),
            full(w0c), full(w0n), full(b0), full(g0), full(be0), full(m0), full(v0),
            full(w1), full(b1), full(g1), full(be1), full(m1), full(v1),
            full(w2), full(b2),
        ],
        out_specs=pl.BlockSpec((TB, 1), row),
        out_shape=jax.ShapeDtypeStruct((B, 1), jnp.float32),
    )(xg, xn, bg, bb, bm, bv, w0c, w0n, b0, g0, be0, m0, v0,
      w1, b1, g1, be1, m1, v1, w2, b2)


def kernel(x_categorical, x_numerical, emb_tables, bn_num_gamma, bn_num_beta,
           bn_num_mean, bn_num_var, w0, b0, g0, be0, m0, v0,
           w1, b1, g1, be1, m1, v1, w2, b2):
    xcat = x_categorical.astype(jnp.int32).T.reshape(F, B // 128, 128)
    table3 = jnp.swapaxes(emb_tables, 1, 2)
    gathered = _sc_gather(table3, xcat)
    xg = gathered.reshape(NP, B)

    r2 = lambda a: a.reshape(1, -1)
    return _tc_mlp(
        xg, x_numerical,
        r2(bn_num_gamma), r2(bn_num_beta), r2(bn_num_mean), r2(bn_num_var),
        w0[:, :NP].T, w0[:, NP:].T, r2(b0), r2(g0), r2(be0), r2(m0), r2(v0),
        w1.T, r2(b1), r2(g1), r2(be1), r2(m1), r2(v1),
        w2.T, r2(b2),
    )
